# baseline (speedup 1.0000x reference)
import jax
import jax.numpy as jnp
from jax.experimental import pallas as pl
from jax.experimental.pallas import tpu as pltpu

T, D, F, E, K = 32, 1024, 704, 64, 8


def _moe_kernel(se_ref, rw_ref, s0_ref, s1_ref, s2_ref, x_ref,
                w0_ref, w1_ref, w2t_ref, o_ref):
    i = pl.program_id(0)
    e = i // 2
    x = x_ref[...].astype(jnp.bfloat16)
    dn_t = (((1,), (1,)), ((), ()))
    dn_n = (((1,), (0,)), ((), ()))
    w0e = w0_ref[0].astype(jnp.bfloat16)
    w1e = w1_ref[0].astype(jnp.bfloat16)
    w2e = w2t_ref[0].astype(jnp.bfloat16)
    g = jax.lax.dot_general(x, w0e, dn_t, preferred_element_type=jnp.float32)
    g = g * s0_ref[e]
    u = jax.lax.dot_general(x, w1e, dn_t, preferred_element_type=jnp.float32)
    h = ((g * jax.nn.sigmoid(g)) * u).astype(jnp.bfloat16)
    y = jax.lax.dot_general(h, w2e, dn_n, preferred_element_type=jnp.float32)
    se = se_ref[...]
    rw = rw_ref[...]
    cw = jnp.sum(jnp.where(se == e, rw, 0.0), axis=1, keepdims=True)
    contrib = y * (cw * (s1_ref[e] * s2_ref[e]))

    @pl.when(i == 0)
    def _():
        o_ref[...] = contrib

    @pl.when(i != 0)
    def _():
        o_ref[...] += contrib


def kernel(x, w0, w1, w2, s0, s1, s2, selected_experts, routing_weights,
           gathered_experts_out_buf, select_experts_middle, routing_weights_middle,
           gather_buffer, scatter_buffer, use_ppl):
    se = selected_experts.astype(jnp.int32)
    w2t = jnp.swapaxes(w2, 1, 2)
    Fh = F // 2
    out = pl.pallas_call(
        _moe_kernel,
        grid=(2 * E,),
        in_specs=[
            pl.BlockSpec((T, K), lambda i: (0, 0)),
            pl.BlockSpec((T, K), lambda i: (0, 0)),
            pl.BlockSpec(memory_space=pltpu.SMEM),
            pl.BlockSpec(memory_space=pltpu.SMEM),
            pl.BlockSpec(memory_space=pltpu.SMEM),
            pl.BlockSpec((T, D), lambda i: (0, 0)),
            pl.BlockSpec((1, Fh, D), lambda i: (i // 2, i % 2, 0)),
            pl.BlockSpec((1, Fh, D), lambda i: (i // 2, i % 2, 0)),
            pl.BlockSpec((1, Fh, D), lambda i: (i // 2, i % 2, 0)),
        ],
        out_specs=pl.BlockSpec((T, D), lambda i: (0, 0)),
        out_shape=jax.ShapeDtypeStruct((T, D), jnp.float32),
    )(se, routing_weights, s0, s1, s2, x, w0, w1, w2t)
    return out


# speedup vs baseline: 1.2480x; 1.2480x over previous
import jax
import jax.numpy as jnp
from jax.experimental import pallas as pl
from jax.experimental.pallas import tpu as pltpu

T, D, F, E, K = 32, 1024, 704, 64, 8


def _moe_kernel(set_ref, rwt_ref, s0_ref, s1_ref, s2_ref, x_ref,
                w0_ref, w1_ref, w2t_ref, o_ref, se_v, rw_v):
    e = pl.program_id(0)

    @pl.when(e == 0)
    def _():
        se_v[...] = jnp.transpose(set_ref[...])
        rw_v[...] = jnp.transpose(rwt_ref[...])

    x = x_ref[...].astype(jnp.bfloat16)
    dn_t = (((1,), (1,)), ((), ()))
    dn_n = (((1,), (0,)), ((), ()))
    w0e = w0_ref[0].astype(jnp.bfloat16)
    w1e = w1_ref[0].astype(jnp.bfloat16)
    w2e = w2t_ref[0].astype(jnp.bfloat16)
    g = jax.lax.dot_general(x, w0e, dn_t, preferred_element_type=jnp.float32)
    g = g * s0_ref[e]
    u = jax.lax.dot_general(x, w1e, dn_t, preferred_element_type=jnp.float32)
    h = ((g * jax.nn.sigmoid(g)) * u).astype(jnp.bfloat16)
    y = jax.lax.dot_general(h, w2e, dn_n, preferred_element_type=jnp.float32)
    cw = jnp.sum(jnp.where(se_v[...] == e, rw_v[...], 0.0),
                 axis=1, keepdims=True)
    contrib = y * (cw * (s1_ref[e] * s2_ref[e]))

    @pl.when(e == 0)
    def _():
        o_ref[...] = contrib

    @pl.when(e != 0)
    def _():
        o_ref[...] += contrib


def kernel(x, w0, w1, w2, s0, s1, s2, selected_experts, routing_weights,
           gathered_experts_out_buf, select_experts_middle, routing_weights_middle,
           gather_buffer, scatter_buffer, use_ppl):
    se_t = jnp.swapaxes(selected_experts.astype(jnp.int32), 0, 1)
    rw_t = jnp.swapaxes(routing_weights, 0, 1)
    w2t = jnp.swapaxes(w2, 1, 2)
    out = pl.pallas_call(
        _moe_kernel,
        grid=(E,),
        in_specs=[
            pl.BlockSpec((K, T), lambda e: (0, 0)),
            pl.BlockSpec((K, T), lambda e: (0, 0)),
            pl.BlockSpec(memory_space=pltpu.SMEM),
            pl.BlockSpec(memory_space=pltpu.SMEM),
            pl.BlockSpec(memory_space=pltpu.SMEM),
            pl.BlockSpec((T, D), lambda e: (0, 0)),
            pl.BlockSpec((1, F, D), lambda e: (e, 0, 0)),
            pl.BlockSpec((1, F, D), lambda e: (e, 0, 0)),
            pl.BlockSpec((1, F, D), lambda e: (e, 0, 0)),
        ],
        out_specs=pl.BlockSpec((T, D), lambda e: (0, 0)),
        out_shape=jax.ShapeDtypeStruct((T, D), jnp.float32),
        scratch_shapes=[
            pltpu.VMEM((T, K), jnp.int32),
            pltpu.VMEM((T, K), jnp.float32),
        ],
    )(se_t, rw_t, s0, s1, s2, x, w0, w1, w2t)
    return out
